# Initial kernel scaffold; baseline (speedup 1.0000x reference)
import jax
import jax.numpy as jnp
from jax.experimental import pallas as pl


def kernel(user_emb, item_emb, edge_index, edge_weight):
    raise NotImplementedError("write your pallas kernel here")


# speedup vs baseline: 1.4368x; 1.4368x over previous
import functools

import jax
import jax.numpy as jnp
from jax import lax
from jax.experimental import pallas as pl
from jax.experimental.pallas import tpu as pltpu
from jax.experimental.pallas import tpu_sc as plsc

N_U = 25000
N_I = 25000
N = N_U + N_I
N_PAD = 50176
D = 64
H = 32
LANES = 16
NS = 16
BLK = 512
NBLK = 98
EDGES_PER_TILE = BLK * NBLK
E_PAD = EDGES_PER_TILE * NS
ROWS_PER_TILE = N_PAD // NS
WCHUNK = 64
NWCHUNK = ROWS_PER_TILE // WCHUNK

_mesh = plsc.VectorSubcoreMesh(core_axis_name="c", subcore_axis_name="s")


@functools.partial(
    pl.kernel,
    out_type=jax.ShapeDtypeStruct((2 * N_PAD, H), jnp.float32),
    mesh=_mesh,
    compiler_params=pltpu.CompilerParams(use_tc_tiling_on_sc=False,
                                         needs_layout_passes=False),
    scratch_types=[
        pltpu.HBM((2 * N_PAD, H), jnp.float32),
        pltpu.HBM((2 * N_PAD, H), jnp.float32),
        pltpu.HBM((2 * N_PAD, H), jnp.float32),
        pltpu.VMEM_SHARED((N_PAD, H), jnp.float32),
        pltpu.VMEM((4, 128), jnp.int32),
        pltpu.VMEM((4, 128), jnp.int32),
        pltpu.VMEM((4, 128), jnp.float32),
        pltpu.VMEM((BLK, H), jnp.float32),
        pltpu.VMEM((WCHUNK, H), jnp.float32),
        pltpu.SemaphoreType.DMA,
    ],
)
def _gcl_kernel(ego0, src2d, dst2d, w2d, out_mean,
                s1, s2, s3, acc, srcb, dstb, wb, rows, zbuf, gsem):
    c = lax.axis_index("c")
    s = lax.axis_index("s")
    coff = c * N_PAD
    row0 = s * ROWS_PER_TILE
    erow0 = s * (EDGES_PER_TILE // 128)

    zero16 = jnp.zeros((LANES,), jnp.float32)
    iota16 = lax.iota(jnp.int32, LANES)

    def zero_zbuf(r, carry):
        zbuf[r, pl.ds(0, 16)] = zero16
        zbuf[r, pl.ds(16, 16)] = zero16
        return carry

    lax.fori_loop(0, WCHUNK, zero_zbuf, 0)

    for t_in, t_out in ((ego0, s1), (s1, s2), (s2, s3)):
        def zero_acc(i, carry):
            pltpu.sync_copy(zbuf, acc.at[pl.ds(row0 + i * WCHUNK, WCHUNK)])
            return carry

        lax.fori_loop(0, NWCHUNK, zero_acc, 0)
        plsc.subcore_barrier()

        def blk_body(b, carry):
            er = erow0 + b * 4
            pltpu.sync_copy(src2d.at[pl.ds(er, 4)], srcb)
            pltpu.sync_copy(dst2d.at[pl.ds(er, 4)], dstb)
            pltpu.sync_copy(w2d.at[pl.ds(er, 4)], wb)

            def add_off(j, carry2):
                for gg in range(8):
                    sl = pl.ds(gg * 16, 16)
                    srcb[j, sl] = srcb[j, sl] + coff
                return carry2

            lax.fori_loop(0, 4, add_off, 0)

            descs = [
                pltpu.async_copy(t_in.at[srcb.at[j]],
                                 rows.at[pl.ds(j * 128, 128)], gsem)
                for j in range(4)
            ]
            for d in descs:
                d.wait()

            def grp(g, carry2):
                jj = g // 8
                ss = (g % 8) * 16
                wv = wb[jj, pl.ds(ss, 16)]
                ridx = g * 16 + iota16
                for col in range(H):
                    cidx = jnp.full((LANES,), col, jnp.int32)
                    v = plsc.load_gather(rows, [ridx, cidx])
                    plsc.store_scatter(rows, [ridx, cidx], v * wv)
                return carry2

            lax.fori_loop(0, BLK // 16, grp, 0)

            sdescs = [
                pltpu.async_copy(rows.at[pl.ds(j * 128, 128)],
                                 acc.at[dstb.at[j]], gsem, add=True)
                for j in range(4)
            ]
            for d in sdescs:
                d.wait()
            return carry

        lax.fori_loop(0, NBLK, blk_body, 0)
        plsc.subcore_barrier()

        def writeout(i, carry):
            r = row0 + i * WCHUNK
            pltpu.sync_copy(acc.at[pl.ds(r, WCHUNK)],
                            t_out.at[pl.ds(coff + r, WCHUNK)])
            return carry

        lax.fori_loop(0, NWCHUNK, writeout, 0)
        plsc.subcore_barrier()

    third = jnp.float32(1.0 / 3.0)

    def mean_chunk(i, carry):
        r = coff + row0 + i * WCHUNK
        pltpu.sync_copy(s1.at[pl.ds(r, WCHUNK)], rows.at[pl.ds(0, WCHUNK)])
        pltpu.sync_copy(s2.at[pl.ds(r, WCHUNK)], rows.at[pl.ds(128, WCHUNK)])
        pltpu.sync_copy(s3.at[pl.ds(r, WCHUNK)], rows.at[pl.ds(256, WCHUNK)])

        def avg_row(rr, carry2):
            for h in (0, 16):
                sl = pl.ds(h, 16)
                zbuf[rr, sl] = (rows[rr, sl] + rows[128 + rr, sl]
                                + rows[256 + rr, sl]) * third
            return carry2

        lax.fori_loop(0, WCHUNK, avg_row, 0)
        pltpu.sync_copy(zbuf, out_mean.at[pl.ds(r, WCHUNK)])
        return carry

    lax.fori_loop(0, NWCHUNK, mean_chunk, 0)


def kernel(user_emb, item_emb, edge_index, edge_weight):
    src = edge_index[0].astype(jnp.int32)
    dst = edge_index[1].astype(jnp.int32)
    w = edge_weight.astype(jnp.float32)

    pad = E_PAD - src.shape[0]
    src = jnp.concatenate([src, jnp.zeros((pad,), jnp.int32)]).reshape(-1, 128)
    dst = jnp.concatenate([dst, jnp.zeros((pad,), jnp.int32)]).reshape(-1, 128)
    w = jnp.concatenate([w, jnp.zeros((pad,), jnp.float32)]).reshape(-1, 128)

    ego0 = jnp.concatenate([user_emb, item_emb], axis=0)
    ego0 = ego0.reshape(N, 2, H).transpose(1, 0, 2)
    ego0 = jnp.pad(ego0, ((0, 0), (0, N_PAD - N), (0, 0))).reshape(2 * N_PAD, H)

    mean_slab = _gcl_kernel(ego0, src, dst, w)

    full = jnp.concatenate(
        [mean_slab[:N], mean_slab[N_PAD:N_PAD + N]], axis=1)
    return full[:N_U], full[N_U:]


# speedup vs baseline: 1.5341x; 1.0677x over previous
import functools

import jax
import jax.numpy as jnp
from jax import lax
from jax.experimental import pallas as pl
from jax.experimental.pallas import tpu as pltpu
from jax.experimental.pallas import tpu_sc as plsc

N_U = 25000
N_I = 25000
N = N_U + N_I
N_PAD = 50176
TBL = 2 * N_PAD
D = 64
H = 32
LANES = 16
NS = 16
STAGE = 128
SB = 4
SB_PROC = 100
SB_ALLOC = 102
IDXROWS = SB_ALLOC * SB
E_ALLOC = IDXROWS * 128 * NS
ROWS_PER_TILE = N_PAD // NS
CCHUNK = 448
DCHUNK = 112

_mesh = plsc.VectorSubcoreMesh(core_axis_name="c", subcore_axis_name="s")


@functools.partial(
    pl.kernel,
    out_type=jax.ShapeDtypeStruct((TBL, H), jnp.float32),
    mesh=_mesh,
    compiler_params=pltpu.CompilerParams(use_tc_tiling_on_sc=False,
                                         needs_layout_passes=False),
    scratch_types=[
        pltpu.HBM((3 * TBL, H), jnp.float32),
        pltpu.VMEM_SHARED((N_PAD, H), jnp.float32),
        pltpu.VMEM((4, STAGE, H), jnp.float32),
        pltpu.VMEM((2, STAGE, H), jnp.float32),
        pltpu.VMEM((2, SB, 128), jnp.int32),
        pltpu.VMEM((2, SB, 128), jnp.int32),
        pltpu.VMEM((2, SB, 128), jnp.int32),
        pltpu.VMEM((2, SB, 128), jnp.float32),
        pltpu.VMEM((32, H), jnp.float32),
        pltpu.SemaphoreType.DMA,
        pltpu.SemaphoreType.DMA,
        pltpu.SemaphoreType.DMA,
        pltpu.SemaphoreType.DMA,
        pltpu.SemaphoreType.DMA,
        pltpu.SemaphoreType.DMA,
        pltpu.SemaphoreType.DMA,
        pltpu.SemaphoreType.DMA,
        pltpu.SemaphoreType.DMA,
        pltpu.SemaphoreType.DMA,
        pltpu.SemaphoreType.DMA,
        pltpu.SemaphoreType.DMA,
        pltpu.SemaphoreType.DMA,
    ],
)
def _gcl_kernel(ego0, src2d, dst2d, w2d, out_mean,
                SL, acc, G, S, srcb, srcoff, dstb, wb, zbuf,
                gsem0, gsem1, gsem2, gsem3, ssem0, ssem1,
                isrc0, isrc1, idst0, idst1, iw0, iw1, csem):
    c = lax.axis_index("c")
    s = lax.axis_index("s")
    coff = c * N_PAD
    row0 = s * ROWS_PER_TILE
    erow0 = s * IDXROWS

    gsems = (gsem0, gsem1, gsem2, gsem3)
    ssems = (ssem0, ssem1)
    isrcs = (isrc0, isrc1)
    idsts = (idst0, idst1)
    iws = (iw0, iw1)
    zero16 = jnp.zeros((LANES,), jnp.float32)
    iota16 = lax.iota(jnp.int32, LANES)
    cols16 = [jnp.full((LANES,), col, jnp.int32) for col in range(H)]
    third = jnp.float32(1.0 / 3.0)

    def zero_zbuf(r, carry):
        zbuf[r, pl.ds(0, 16)] = zero16
        zbuf[r, pl.ds(16, 16)] = zero16
        return carry

    lax.fori_loop(0, 32, zero_zbuf, 0)

    sbase = (c * NS + s) * (TBL // (2 * NS))
    in_descs = [
        pltpu.async_copy(ego0.at[pl.ds(sbase + i * CCHUNK, CCHUNK)],
                         SL.at[pl.ds(sbase + i * CCHUNK, CCHUNK)], csem)
        for i in range(TBL // (2 * NS) // CCHUNK)
    ]
    for d in in_descs:
        d.wait()
    plsc.subcore_barrier()

    def offset_add(par, gbase):
        for r in range(SB):
            for h in range(8):
                sl = pl.ds(h * 16, 16)
                srcoff[par, r, sl] = srcb[par, r, sl] + gbase

    def fire_src(sb2, par):
        er = erow0 + sb2 * SB
        pltpu.async_copy(src2d.at[pl.ds(er, SB)], srcb.at[par], isrcs[par])

    def drain_src(par):
        pltpu.make_async_copy(src2d.at[pl.ds(0, SB)], srcb.at[par],
                              isrcs[par]).wait()

    def fire_dst(sb2, par):
        er = erow0 + sb2 * SB
        pltpu.async_copy(dst2d.at[pl.ds(er, SB)], dstb.at[par], idsts[par])

    def drain_dst(par):
        pltpu.make_async_copy(dst2d.at[pl.ds(0, SB)], dstb.at[par],
                              idsts[par]).wait()

    def fire_w(sb2, par):
        er = erow0 + sb2 * SB
        pltpu.async_copy(w2d.at[pl.ds(er, SB)], wb.at[par], iws[par])

    def drain_w(par):
        pltpu.make_async_copy(w2d.at[pl.ds(0, SB)], wb.at[par],
                              iws[par]).wait()

    def fire_gather(t, par):
        pltpu.async_copy(SL.at[srcoff.at[par, t]], G.at[t], gsems[t])

    def drain_gather(t):
        pltpu.make_async_copy(SL.at[srcoff.at[0, t]], G.at[t],
                              gsems[t]).wait()

    def fire_scatter(t, par):
        pltpu.async_copy(S.at[t % 2], acc.at[dstb.at[par, t]],
                         ssems[t % 2], add=True)

    def drain_scatter(t):
        pltpu.make_async_copy(S.at[t % 2], acc.at[dstb.at[0, t]],
                              ssems[t % 2]).wait()

    def scale(t, par):
        Gt = G.at[t]
        St = S.at[t % 2]

        def grp(g, carry):
            wv = wb[par, t, pl.ds(g * 16, 16)]
            ridx = g * 16 + iota16
            for col in range(H):
                v = plsc.load_gather(Gt, [ridx, cols16[col]])
                plsc.store_scatter(St, [ridx, cols16[col]], v * wv)
            return carry

        lax.fori_loop(0, STAGE // 16, grp, 0)

    def layer_body(k, carry):
        gbase = coff + k * TBL

        def zero_acc(i, carry2):
            pltpu.sync_copy(zbuf, acc.at[pl.ds(row0 + i * 32, 32)])
            return carry2

        lax.fori_loop(0, ROWS_PER_TILE // 32, zero_acc, 0)
        plsc.subcore_barrier()

        pltpu.sync_copy(src2d.at[pl.ds(erow0, SB)], srcb.at[0])
        offset_add(0, gbase)
        fire_src(1, 1)
        fire_w(0, 0)
        fire_w(1, 1)
        fire_dst(0, 0)
        for t in range(SB):
            fire_gather(t, 0)

        def body(sbv, par, guard_first):
            drain_w(par)
            drain_dst(par)
            drain_src(1 - par)
            offset_add(1 - par, gbase)
            fire_src(sbv + 2, par)
            for t in range(SB):
                drain_gather(t)
                if guard_first and t < 2:
                    @pl.when(sbv > 0)
                    def _(t=t):
                        drain_scatter(t)
                else:
                    drain_scatter(t)
                scale(t, par)
                fire_scatter(t, par)
                fire_gather(t, 1 - par)
                if t == 1:
                    fire_dst(sbv + 1, 1 - par)
            fire_w(sbv + 2, par)

        def loop_u(u, carry2):
            sb0 = 2 * u
            body(sb0, 0, True)
            body(sb0 + 1, 1, False)
            return carry2

        lax.fori_loop(0, SB_PROC // 2, loop_u, 0)

        for t in range(SB):
            drain_gather(t)
        drain_scatter(0)
        drain_scatter(1)
        drain_src(1)
        drain_dst(0)
        drain_w(0)
        drain_w(1)
        plsc.subcore_barrier()

        @pl.when(k < 2)
        def _():
            obase = (k + 1) * TBL + coff
            descs = [
                pltpu.async_copy(
                    acc.at[pl.ds(row0 + i * CCHUNK, CCHUNK)],
                    SL.at[pl.ds(obase + row0 + i * CCHUNK, CCHUNK)], csem)
                for i in range(ROWS_PER_TILE // CCHUNK)
            ]
            for d in descs:
                d.wait()

        plsc.subcore_barrier()
        return carry

    lax.fori_loop(0, 3, layer_body, 0)

    def mean_chunk(i, carry):
        r = row0 + i * DCHUNK
        d1 = pltpu.async_copy(SL.at[pl.ds(TBL + coff + r, DCHUNK)],
                              S.at[0, pl.ds(0, DCHUNK)], gsem0)
        d2 = pltpu.async_copy(SL.at[pl.ds(2 * TBL + coff + r, DCHUNK)],
                              S.at[1, pl.ds(0, DCHUNK)], gsem1)
        pltpu.sync_copy(acc.at[pl.ds(r, DCHUNK)], G.at[0, pl.ds(0, DCHUNK)])
        d1.wait()
        d2.wait()

        def avg_row(rr, carry2):
            for h in (0, 16):
                sl = pl.ds(h, 16)
                G[1, rr, sl] = (S[0, rr, sl] + S[1, rr, sl]
                                + G[0, rr, sl]) * third
            return carry2

        lax.fori_loop(0, DCHUNK, avg_row, 0)
        pltpu.sync_copy(G.at[1, pl.ds(0, DCHUNK)],
                        out_mean.at[pl.ds(coff + r, DCHUNK)])
        return carry

    lax.fori_loop(0, ROWS_PER_TILE // DCHUNK, mean_chunk, 0)


def kernel(user_emb, item_emb, edge_index, edge_weight):
    src = edge_index[0].astype(jnp.int32)
    dst = edge_index[1].astype(jnp.int32)
    w = edge_weight.astype(jnp.float32)

    nproc = SB_PROC * SB * 128
    pad = NS * nproc - src.shape[0]

    def layout(x, pv):
        x = jnp.concatenate([x, jnp.full((pad,), pv, x.dtype)])
        x = x.reshape(NS, SB_PROC * SB, 128)
        tail = jnp.zeros((NS, (SB_ALLOC - SB_PROC) * SB, 128), x.dtype)
        return jnp.concatenate([x, tail], axis=1).reshape(-1, 128)

    src = layout(src, 0)
    dst = layout(dst, 0)
    w = layout(w, 0.0)

    ego0 = jnp.concatenate([user_emb, item_emb], axis=0)
    ego0 = ego0.reshape(N, 2, H).transpose(1, 0, 2)
    ego0 = jnp.pad(ego0, ((0, 0), (0, N_PAD - N), (0, 0))).reshape(TBL, H)

    mean_slab = _gcl_kernel(ego0, src, dst, w)

    full = jnp.concatenate(
        [mean_slab[:N], mean_slab[N_PAD:N_PAD + N]], axis=1)
    return full[:N_U], full[N_U:]
